# Initial kernel scaffold; baseline (speedup 1.0000x reference)
import jax
import jax.numpy as jnp
from jax.experimental import pallas as pl


def kernel(pred, target):
    raise NotImplementedError("write your pallas kernel here")


# speedup vs baseline: 1.5173x; 1.5173x over previous
import functools

import jax
import jax.numpy as jnp
from jax import lax
from jax.experimental import pallas as pl
from jax.experimental.pallas import tpu as pltpu

_SMOOTH = 0.1
_CONF = 1.0 - _SMOOTH
_IGN = 0


def _tc_body(pred_ref, tgt_ref, out_ref, macc, sacc, spacc, ptacc, p0,
             *, C, R, W, NROW):
    r = pl.program_id(0)
    j = pl.program_id(1)
    nr = pl.num_programs(0)
    nj = pl.num_programs(1)
    NK = W // 128

    @pl.when(j == 0)
    def _init():
        macc[...] = jnp.full((R, 128), -jnp.inf, jnp.float32)
        sacc[...] = jnp.zeros((R, 128), jnp.float32)
        spacc[...] = jnp.zeros((R, 128), jnp.float32)
        ptacc[...] = jnp.zeros((R, 128), jnp.float32)
        p0[...] = pred_ref[:, 0:1]

    t = tgt_ref[...]

    def process(masked):
        x = pred_ref[...].reshape(R, NK, 128)
        col = (j * W
               + lax.broadcasted_iota(jnp.int32, (R, NK, 128), 1) * 128
               + lax.broadcasted_iota(jnp.int32, (R, NK, 128), 2))
        if masked:
            valid = col < C
            xm = jnp.where(valid, x, -jnp.inf)
            xs = jnp.where(valid, x, 0.0)
        else:
            xm = x
            xs = x
        bm = jnp.max(xm, axis=1)
        mnew = jnp.maximum(macc[...], bm)
        scale = jnp.exp(macc[...] - mnew)
        e = jnp.exp(xm - mnew[:, None, :])
        sacc[...] = sacc[...] * scale + jnp.sum(e, axis=1)
        macc[...] = mnew
        spacc[...] = spacc[...] + jnp.sum(xs, axis=1)
        hit = col == t[:, :, None]
        ptacc[...] = ptacc[...] + jnp.sum(jnp.where(hit, x, 0.0), axis=1)

    @pl.when(j < nj - 1)
    def _main():
        process(False)

    @pl.when(j == nj - 1)
    def _fin():
        process(True)
        M = jnp.max(macc[...], axis=1, keepdims=True)
        S = jnp.sum(sacc[...] * jnp.exp(macc[...] - M), axis=1,
                    keepdims=True)
        SP = jnp.sum(spacc[...], axis=1, keepdims=True)
        PT = jnp.sum(ptacc[...], axis=1, keepdims=True)
        lse = M + jnp.log(S)
        logpt = PT - lse
        logp0 = p0[...] - lse
        ssum = SP - C * lse
        row = (_SMOOTH / (C - 1)) * (ssum - logp0 - logpt) + _CONF * logpt
        row = jnp.where(t == _IGN, 0.0, -row)
        bsum = jnp.sum(row)
        tot = jnp.where(r == 0, bsum, out_ref[0, 0] + bsum)
        out_ref[0, 0] = jnp.where(r == nr - 1, tot / NROW, tot)


def kernel(pred, target):
    N, C = pred.shape
    R, W = 256, 2048
    nj = -(-C // W)
    t2 = target.astype(jnp.int32).reshape(N, 1)
    out = pl.pallas_call(
        functools.partial(_tc_body, C=C, R=R, W=W, NROW=N),
        grid=(N // R, nj),
        in_specs=[
            pl.BlockSpec((R, W), lambda r, j: (r, j)),
            pl.BlockSpec((R, 1), lambda r, j: (r, 0)),
        ],
        out_specs=pl.BlockSpec((1, 1), lambda r, j: (0, 0),
                               memory_space=pltpu.SMEM),
        out_shape=jax.ShapeDtypeStruct((1, 1), jnp.float32),
        scratch_shapes=[
            pltpu.VMEM((R, 128), jnp.float32),
            pltpu.VMEM((R, 128), jnp.float32),
            pltpu.VMEM((R, 128), jnp.float32),
            pltpu.VMEM((R, 128), jnp.float32),
            pltpu.VMEM((R, 1), jnp.float32),
        ],
        compiler_params=pltpu.CompilerParams(
            dimension_semantics=("arbitrary", "arbitrary")),
    )(pred, t2)
    return out[0, 0]


# speedup vs baseline: 1.6477x; 1.0860x over previous
import functools

import jax
import jax.numpy as jnp
from jax import lax
from jax.experimental import pallas as pl
from jax.experimental.pallas import tpu as pltpu

_SMOOTH = 0.1
_CONF = 1.0 - _SMOOTH
_IGN = 0


def _tc_body(pred_ref, tgt_ref, out_ref, macc, sacc, spacc, ptacc, p0,
             *, C, R, W, NROW):
    r = pl.program_id(0)
    j = pl.program_id(1)
    nr = pl.num_programs(0)
    nj = pl.num_programs(1)
    NK = W // 128

    @pl.when(j == 0)
    def _init():
        macc[...] = jnp.full((R, 128), -jnp.inf, jnp.float32)
        sacc[...] = jnp.zeros((R, 128), jnp.float32)
        spacc[...] = jnp.zeros((R, 128), jnp.float32)
        ptacc[...] = jnp.zeros((R, 128), jnp.float32)
        p0[...] = pred_ref[:, 0:1]

    t = tgt_ref[...]
    lane8 = lax.broadcasted_iota(jnp.int32, (8, 128), 1)

    def process(nk_full, part_lanes):
        nk = nk_full + (1 if part_lanes else 0)
        for ir in range(R // 8):
            sl = slice(ir * 8, (ir + 1) * 8)
            xs = []
            for k in range(nk):
                x = pred_ref[sl, k * 128:(k + 1) * 128]
                if part_lanes and k == nk_full:
                    x = jnp.where(lane8 < part_lanes, x, -jnp.inf)
                xs.append(x)
            bm = xs[0]
            for k in range(1, nk):
                bm = jnp.maximum(bm, xs[k])
            m_old = macc[sl, :]
            mnew = jnp.maximum(m_old, bm)
            scale = jnp.exp(m_old - mnew)
            macc[sl, :] = mnew

            tb = jnp.broadcast_to(t[sl, :], (8, 128)) - (j * W + lane8)
            se0 = jnp.exp(xs[0] - mnew)
            se1 = jnp.zeros((8, 128), jnp.float32)
            sp = xs[0]
            pt = jnp.where(tb == 0, xs[0], 0.0)
            for k in range(1, nk):
                x = xs[k]
                e = jnp.exp(x - mnew)
                if k % 2 == 0:
                    se0 = se0 + e
                else:
                    se1 = se1 + e
                if part_lanes and k == nk_full:
                    x = jnp.where(lane8 < part_lanes, x, 0.0)
                sp = sp + x
                pt = pt + jnp.where(tb == k * 128, x, 0.0)
            sacc[sl, :] = sacc[sl, :] * scale + (se0 + se1)
            spacc[sl, :] = spacc[sl, :] + sp
            ptacc[sl, :] = ptacc[sl, :] + pt

    tail = C - (nj - 1) * W

    @pl.when(j < nj - 1)
    def _main():
        process(NK, 0)

    @pl.when(j == nj - 1)
    def _fin():
        process(tail // 128, tail % 128)
        M = jnp.max(macc[...], axis=1, keepdims=True)
        S = jnp.sum(sacc[...] * jnp.exp(macc[...] - M), axis=1,
                    keepdims=True)
        SP = jnp.sum(spacc[...], axis=1, keepdims=True)
        PT = jnp.sum(ptacc[...], axis=1, keepdims=True)
        lse = M + jnp.log(S)
        logpt = PT - lse
        logp0 = p0[...] - lse
        ssum = SP - C * lse
        row = (_SMOOTH / (C - 1)) * (ssum - logp0 - logpt) + _CONF * logpt
        row = jnp.where(t == _IGN, 0.0, -row)
        bsum = jnp.sum(row)
        tot = jnp.where(r == 0, bsum, out_ref[0, 0] + bsum)
        out_ref[0, 0] = jnp.where(r == nr - 1, tot / NROW, tot)


def kernel(pred, target):
    N, C = pred.shape
    R, W = 128, 2048
    nj = -(-C // W)
    t2 = target.astype(jnp.int32).reshape(N, 1)
    out = pl.pallas_call(
        functools.partial(_tc_body, C=C, R=R, W=W, NROW=N),
        grid=(N // R, nj),
        in_specs=[
            pl.BlockSpec((R, W), lambda r, j: (r, j)),
            pl.BlockSpec((R, 1), lambda r, j: (r, 0)),
        ],
        out_specs=pl.BlockSpec((1, 1), lambda r, j: (0, 0),
                               memory_space=pltpu.SMEM),
        out_shape=jax.ShapeDtypeStruct((1, 1), jnp.float32),
        scratch_shapes=[
            pltpu.VMEM((R, 128), jnp.float32),
            pltpu.VMEM((R, 128), jnp.float32),
            pltpu.VMEM((R, 128), jnp.float32),
            pltpu.VMEM((R, 128), jnp.float32),
            pltpu.VMEM((R, 1), jnp.float32),
        ],
        compiler_params=pltpu.CompilerParams(
            dimension_semantics=("arbitrary", "arbitrary")),
    )(pred, t2)
    return out[0, 0]


# speedup vs baseline: 2.1048x; 1.2774x over previous
import functools

import jax
import jax.numpy as jnp
from jax import lax
from jax.experimental import pallas as pl
from jax.experimental.pallas import tpu as pltpu
from jax.experimental.pallas import tpu_sc as plsc

_SMOOTH = 0.1
_CONF = 1.0 - _SMOOTH
_IGN = 0


def _sc_gather_call(pred, target):
    N = pred.shape[0]
    info = plsc.get_sparse_core_info()
    nw = info.num_cores * info.num_subcores
    bpw = N // nw
    mesh = plsc.VectorSubcoreMesh(core_axis_name="c", subcore_axis_name="s")

    @functools.partial(
        pl.kernel, mesh=mesh,
        out_type=jax.ShapeDtypeStruct((N, 1), jnp.float32),
        scratch_types=[
            pltpu.VMEM((bpw,), jnp.int32),
            pltpu.VMEM((bpw, 1), jnp.float32),
            pltpu.SemaphoreType.DMA,
        ],
    )
    def sc_gather(pred_hbm, tgt_hbm, out_hbm, tgt_v, val_v, sem):
        wid = lax.axis_index("s") * info.num_cores + lax.axis_index("c")
        base = wid * bpw
        pltpu.sync_copy(tgt_hbm.at[pl.ds(base, bpw)], tgt_v)
        cps = [
            pltpu.async_copy(
                pred_hbm.at[base + i, pl.ds(tgt_v[i], 1)], val_v.at[i], sem)
            for i in range(bpw)
        ]
        for cp in cps:
            cp.wait()
        pltpu.sync_copy(val_v, out_hbm.at[pl.ds(base, bpw)])

    return sc_gather(pred, target)


def _tc_body(pred_ref, tgt_ref, out_ref, macc, sacc, spacc, ptacc, p0,
             *, C, R, W, NROW):
    r = pl.program_id(0)
    j = pl.program_id(1)
    nr = pl.num_programs(0)
    nj = pl.num_programs(1)
    NK = W // 128

    @pl.when(j == 0)
    def _init():
        macc[...] = jnp.full((R, 128), -jnp.inf, jnp.float32)
        sacc[...] = jnp.zeros((R, 128), jnp.float32)
        spacc[...] = jnp.zeros((R, 128), jnp.float32)
        ptacc[...] = jnp.zeros((R, 128), jnp.float32)
        p0[...] = pred_ref[:, 0:1]

    t = tgt_ref[...]
    lane8 = lax.broadcasted_iota(jnp.int32, (8, 128), 1)

    def process(nk_full, part_lanes):
        nk = nk_full + (1 if part_lanes else 0)
        for ir in range(R // 8):
            sl = slice(ir * 8, (ir + 1) * 8)
            xs = []
            for k in range(nk):
                x = pred_ref[sl, k * 128:(k + 1) * 128]
                if part_lanes and k == nk_full:
                    x = jnp.where(lane8 < part_lanes, x, -jnp.inf)
                xs.append(x)
            bm = xs[0]
            for k in range(1, nk):
                bm = jnp.maximum(bm, xs[k])
            m_old = macc[sl, :]
            mnew = jnp.maximum(m_old, bm)
            scale = jnp.exp(m_old - mnew)
            macc[sl, :] = mnew

            tb = jnp.broadcast_to(t[sl, :], (8, 128)) - (j * W + lane8)
            se0 = jnp.exp(xs[0] - mnew)
            se1 = jnp.zeros((8, 128), jnp.float32)
            sp = xs[0]
            pt = jnp.where(tb == 0, xs[0], 0.0)
            for k in range(1, nk):
                x = xs[k]
                e = jnp.exp(x - mnew)
                if k % 2 == 0:
                    se0 = se0 + e
                else:
                    se1 = se1 + e
                if part_lanes and k == nk_full:
                    x = jnp.where(lane8 < part_lanes, x, 0.0)
                sp = sp + x
                pt = pt + jnp.where(tb == k * 128, x, 0.0)
            sacc[sl, :] = sacc[sl, :] * scale + (se0 + se1)
            spacc[sl, :] = spacc[sl, :] + sp
            ptacc[sl, :] = ptacc[sl, :] + pt

    tail = C - (nj - 1) * W

    @pl.when(j < nj - 1)
    def _main():
        process(NK, 0)

    @pl.when(j == nj - 1)
    def _fin():
        process(tail // 128, tail % 128)
        M = jnp.max(macc[...], axis=1, keepdims=True)
        S = jnp.sum(sacc[...] * jnp.exp(macc[...] - M), axis=1,
                    keepdims=True)
        SP = jnp.sum(spacc[...], axis=1, keepdims=True)
        PT = jnp.sum(ptacc[...], axis=1, keepdims=True)
        lse = M + jnp.log(S)
        logpt = PT - lse
        logp0 = p0[...] - lse
        ssum = SP - C * lse
        row = (_SMOOTH / (C - 1)) * (ssum - logp0 - logpt) + _CONF * logpt
        row = jnp.where(t == _IGN, 0.0, -row)
        bsum = jnp.sum(row)
        tot = jnp.where(r == 0, bsum, out_ref[0, 0] + bsum)
        out_ref[0, 0] = jnp.where(r == nr - 1, tot / NROW, tot)


def kernel(pred, target):
    N, C = pred.shape
    R, W = 256, 4096
    nj = -(-C // W)
    t2 = target.astype(jnp.int32).reshape(N, 1)
    out = pl.pallas_call(
        functools.partial(_tc_body, C=C, R=R, W=W, NROW=N),
        grid=(N // R, nj),
        in_specs=[
            pl.BlockSpec((R, W), lambda r, j: (r, j)),
            pl.BlockSpec((R, 1), lambda r, j: (r, 0)),
        ],
        out_specs=pl.BlockSpec((1, 1), lambda r, j: (0, 0),
                               memory_space=pltpu.SMEM),
        out_shape=jax.ShapeDtypeStruct((1, 1), jnp.float32),
        scratch_shapes=[
            pltpu.VMEM((R, 128), jnp.float32),
            pltpu.VMEM((R, 128), jnp.float32),
            pltpu.VMEM((R, 128), jnp.float32),
            pltpu.VMEM((R, 128), jnp.float32),
            pltpu.VMEM((R, 1), jnp.float32),
        ],
        compiler_params=pltpu.CompilerParams(
            dimension_semantics=("arbitrary", "arbitrary")),
    )(pred, t2)
    return out[0, 0]


# speedup vs baseline: 2.2030x; 1.0466x over previous
import functools

import jax
import jax.numpy as jnp
from jax import lax
from jax.experimental import pallas as pl
from jax.experimental.pallas import tpu as pltpu
from jax.experimental.pallas import tpu_sc as plsc

_SMOOTH = 0.1
_CONF = 1.0 - _SMOOTH
_IGN = 0


def _sc_gather_call(pred, target):
    N = pred.shape[0]
    info = plsc.get_sparse_core_info()
    nw = info.num_cores * info.num_subcores
    bpw = N // nw
    mesh = plsc.VectorSubcoreMesh(core_axis_name="c", subcore_axis_name="s")

    @functools.partial(
        pl.kernel, mesh=mesh,
        out_type=jax.ShapeDtypeStruct((N, 1), jnp.float32),
        scratch_types=[
            pltpu.VMEM((bpw,), jnp.int32),
            pltpu.VMEM((bpw, 1), jnp.float32),
            pltpu.SemaphoreType.DMA,
        ],
    )
    def sc_gather(pred_hbm, tgt_hbm, out_hbm, tgt_v, val_v, sem):
        wid = lax.axis_index("s") * info.num_cores + lax.axis_index("c")
        base = wid * bpw
        pltpu.sync_copy(tgt_hbm.at[pl.ds(base, bpw)], tgt_v)
        cps = [
            pltpu.async_copy(
                pred_hbm.at[base + i, pl.ds(tgt_v[i], 1)], val_v.at[i], sem)
            for i in range(bpw)
        ]
        for cp in cps:
            cp.wait()
        pltpu.sync_copy(val_v, out_hbm.at[pl.ds(base, bpw)])

    return sc_gather(pred, target)


def _tc_body(pred_ref, tgt_ref, out_ref, macc, sacc, spacc, ptacc, p0,
             *, C, R, W, NROW):
    r = pl.program_id(0)
    j = pl.program_id(1)
    nr = pl.num_programs(0)
    nj = pl.num_programs(1)
    NK = W // 128

    @pl.when(j == 0)
    def _init():
        macc[...] = jnp.full((R, 128), -jnp.inf, jnp.float32)
        sacc[...] = jnp.zeros((R, 128), jnp.float32)
        spacc[...] = jnp.zeros((R, 128), jnp.float32)
        ptacc[...] = jnp.zeros((R, 128), jnp.float32)
        p0[...] = pred_ref[:, 0:1]

    t = tgt_ref[...]
    lane8 = lax.broadcasted_iota(jnp.int32, (8, 128), 1)

    def process(nk_full, part_lanes):
        nk = nk_full + (1 if part_lanes else 0)
        for ir in range(R // 8):
            sl = slice(ir * 8, (ir + 1) * 8)
            xs = []
            for k in range(nk):
                x = pred_ref[sl, k * 128:(k + 1) * 128]
                if part_lanes and k == nk_full:
                    x = jnp.where(lane8 < part_lanes, x, -jnp.inf)
                xs.append(x)
            bm = xs[0]
            for k in range(1, nk):
                bm = jnp.maximum(bm, xs[k])
            m_old = macc[sl, :]
            mnew = jnp.maximum(m_old, bm)
            scale = jnp.exp(m_old - mnew)
            macc[sl, :] = mnew

            tb = jnp.broadcast_to(t[sl, :], (8, 128)) - (j * W + lane8)
            se0 = jnp.exp(xs[0] - mnew)
            se1 = jnp.zeros((8, 128), jnp.float32)
            sp = xs[0]
            pt = jnp.where(tb == 0, xs[0], 0.0)
            for k in range(1, nk):
                x = xs[k]
                e = jnp.exp(x - mnew)
                if k % 2 == 0:
                    se0 = se0 + e
                else:
                    se1 = se1 + e
                if part_lanes and k == nk_full:
                    x = jnp.where(lane8 < part_lanes, x, 0.0)
                sp = sp + x
                pt = pt + jnp.where(tb == k * 128, x, 0.0)
            sacc[sl, :] = sacc[sl, :] * scale + (se0 + se1)
            spacc[sl, :] = spacc[sl, :] + sp
            ptacc[sl, :] = ptacc[sl, :] + pt

    tail = C - (nj - 1) * W

    @pl.when(j < nj - 1)
    def _main():
        process(NK, 0)

    @pl.when(j == nj - 1)
    def _fin():
        process(tail // 128, tail % 128)
        M = jnp.max(macc[...], axis=1, keepdims=True)
        S = jnp.sum(sacc[...] * jnp.exp(macc[...] - M), axis=1,
                    keepdims=True)
        SP = jnp.sum(spacc[...], axis=1, keepdims=True)
        PT = jnp.sum(ptacc[...], axis=1, keepdims=True)
        lse = M + jnp.log(S)
        logpt = PT - lse
        logp0 = p0[...] - lse
        ssum = SP - C * lse
        row = (_SMOOTH / (C - 1)) * (ssum - logp0 - logpt) + _CONF * logpt
        row = jnp.where(t == _IGN, 0.0, -row)
        bsum = jnp.sum(row)
        tot = jnp.where(r == 0, bsum, out_ref[0, 0] + bsum)
        out_ref[0, 0] = jnp.where(r == nr - 1, tot / NROW, tot)


def kernel(pred, target):
    N, C = pred.shape
    R, W = 256, 8192
    nj = -(-C // W)
    t2 = target.astype(jnp.int32).reshape(N, 1)
    out = pl.pallas_call(
        functools.partial(_tc_body, C=C, R=R, W=W, NROW=N),
        grid=(N // R, nj),
        in_specs=[
            pl.BlockSpec((R, W), lambda r, j: (r, j)),
            pl.BlockSpec((R, 1), lambda r, j: (r, 0)),
        ],
        out_specs=pl.BlockSpec((1, 1), lambda r, j: (0, 0),
                               memory_space=pltpu.SMEM),
        out_shape=jax.ShapeDtypeStruct((1, 1), jnp.float32),
        scratch_shapes=[
            pltpu.VMEM((R, 128), jnp.float32),
            pltpu.VMEM((R, 128), jnp.float32),
            pltpu.VMEM((R, 128), jnp.float32),
            pltpu.VMEM((R, 128), jnp.float32),
            pltpu.VMEM((R, 1), jnp.float32),
        ],
        compiler_params=pltpu.CompilerParams(
            dimension_semantics=("arbitrary", "arbitrary")),
    )(pred, t2)
    return out[0, 0]


# speedup vs baseline: 7.0575x; 3.2036x over previous
import functools

import jax
import jax.numpy as jnp
from jax import lax
from jax.experimental import pallas as pl
from jax.experimental.pallas import tpu as pltpu

_SMOOTH = 0.1
_CONF = 1.0 - _SMOOTH
_IGN = 0


def _tc_body(predT_ref, tgt_ref, out_ref, macc, sacc, spacc, ptacc, p0,
             *, C, N, H):
    j = pl.program_id(0)
    nj = pl.num_programs(0)
    G = H // 8
    sub8 = lax.broadcasted_iota(jnp.int32, (8, N), 0)

    @pl.when(j == 0)
    def _init():
        macc[...] = jnp.full((8, N), -jnp.inf, jnp.float32)
        sacc[...] = jnp.zeros((8, N), jnp.float32)
        spacc[...] = jnp.zeros((8, N), jnp.float32)
        ptacc[...] = jnp.zeros((8, N), jnp.float32)
        p0[...] = predT_ref[0:1, :]

    t = tgt_ref[...]
    sub8c = lax.broadcasted_iota(jnp.int32, (8, 128), 0)

    def process(ng, rem):
        ngt = ng + (1 if rem else 0)
        for c in range(N // 128):
            cs = slice(c * 128, (c + 1) * 128)

            def load(g):
                x = predT_ref[g * 8:(g + 1) * 8, cs]
                if rem and g == ng:
                    x = jnp.where(sub8c < rem, x, -jnp.inf)
                return x

            bm = load(0)
            for g in range(1, ngt):
                bm = jnp.maximum(bm, load(g))
            m_old = macc[:, cs]
            mnew = jnp.maximum(m_old, bm)
            scale = jnp.exp(m_old - mnew)
            macc[:, cs] = mnew

            tjs = (jnp.broadcast_to(t[:, cs], (8, 128))
                   - (j * H) - sub8c)
            se0 = jnp.zeros((8, 128), jnp.float32)
            se1 = jnp.zeros((8, 128), jnp.float32)
            sp = jnp.zeros((8, 128), jnp.float32)
            pt = jnp.zeros((8, 128), jnp.float32)
            for g in range(ngt):
                x = load(g)
                e = jnp.exp(x - mnew)
                if g % 2 == 0:
                    se0 = se0 + e
                else:
                    se1 = se1 + e
                if rem and g == ng:
                    x = jnp.where(sub8c < rem, x, 0.0)
                sp = sp + x
                pt = pt + jnp.where(tjs == g * 8, x, 0.0)
            sacc[:, cs] = sacc[:, cs] * scale + (se0 + se1)
            spacc[:, cs] = spacc[:, cs] + sp
            ptacc[:, cs] = ptacc[:, cs] + pt

    tail = C - (nj - 1) * H

    @pl.when(j < nj - 1)
    def _main():
        process(G, 0)

    @pl.when(j == nj - 1)
    def _fin():
        process(tail // 8, tail % 8)
        M = jnp.max(macc[...], axis=0, keepdims=True)
        S = jnp.sum(sacc[...] * jnp.exp(macc[...] - M), axis=0,
                    keepdims=True)
        SP = jnp.sum(spacc[...], axis=0, keepdims=True)
        PT = jnp.sum(ptacc[...], axis=0, keepdims=True)
        lse = M + jnp.log(S)
        logpt = PT - lse
        logp0 = p0[...] - lse
        ssum = SP - C * lse
        row = (_SMOOTH / (C - 1)) * (ssum - logp0 - logpt) + _CONF * logpt
        row = jnp.where(t == _IGN, 0.0, -row)
        out_ref[0, 0] = jnp.sum(row) / N


def kernel(pred, target):
    N, C = pred.shape
    H = 2048
    nj = -(-C // H)
    predT = pred.T
    t2 = target.astype(jnp.int32).reshape(1, N)
    out = pl.pallas_call(
        functools.partial(_tc_body, C=C, N=N, H=H),
        grid=(nj,),
        in_specs=[
            pl.BlockSpec((H, N), lambda j: (j, 0)),
            pl.BlockSpec((1, N), lambda j: (0, 0)),
        ],
        out_specs=pl.BlockSpec((1, 1), lambda j: (0, 0),
                               memory_space=pltpu.SMEM),
        out_shape=jax.ShapeDtypeStruct((1, 1), jnp.float32),
        scratch_shapes=[
            pltpu.VMEM((8, N), jnp.float32),
            pltpu.VMEM((8, N), jnp.float32),
            pltpu.VMEM((8, N), jnp.float32),
            pltpu.VMEM((8, N), jnp.float32),
            pltpu.VMEM((1, N), jnp.float32),
        ],
        compiler_params=pltpu.CompilerParams(
            dimension_semantics=("arbitrary",)),
    )(predT, t2)
    return out[0, 0]
